# baseline (speedup 1.0000x reference)
import functools

import jax
import jax.numpy as jnp
from jax import lax
from jax.experimental import pallas as pl
from jax.experimental.pallas import tpu as pltpu
from jax.experimental.pallas import tpu_sc as plsc

CROP = 14
NPIX = CROP * CROP
NC, NS, L = 2, 16, 16
NW = NC * NS
NP = 17
NXP = 17
CH = 128
XP = 15
CHXP = CH * XP
NPCH = NXP * CH
PWORDS = NP * NPCH
OWORDS = CH * NPIX


def _sc_roialign(tbl, rec_i, rec_f, *, m, c, nhw, w):
    bpw = m // NW
    halves = c // CH
    total = bpw * halves

    @functools.partial(
        pl.kernel,
        out_type=jax.ShapeDtypeStruct((m, c * NPIX), jnp.float32),
        mesh=plsc.VectorSubcoreMesh(core_axis_name="c", subcore_axis_name="s"),
        scratch_types=[
            pltpu.VMEM((bpw * 128,), jnp.int32),
            pltpu.VMEM((bpw * 128,), jnp.float32),
            pltpu.VMEM((PWORDS,), jnp.float32),
            pltpu.VMEM((NP * CHXP + 16,), jnp.float32),
            pltpu.VMEM((CH * NPIX,), jnp.float32),
            pltpu.SemaphoreType.DMA,
            pltpu.SemaphoreType.DMA,
        ],
        compiler_params=pltpu.CompilerParams(needs_layout_passes=False),
    )
    def k(tbl_hbm, reci_hbm, recf_hbm, out_hbm,
          reci_v, recf_v, patch_v, xrow_v, out_v, sem, osem):
        wid = lax.axis_index("s") * NC + lax.axis_index("c")
        lane = lax.iota(jnp.int32, L)
        jmask = lane < CROP
        lane_xp = lane * XP

        for bb in range(bpw):
            pltpu.sync_copy(reci_hbm.at[wid * bpw + bb],
                            reci_v.at[pl.ds(bb * 128, 128)])
            pltpu.sync_copy(recf_hbm.at[wid * bpw + bb],
                            recf_v.at[pl.ds(bb * 128, 128)])

        def slab_copies(pp, make_only):
            bx = jnp.right_shift(pp, 1)
            half = jnp.bitwise_and(pp, halves - 1)
            row_base = reci_v[pl.ds(bx * 128 + 64, L)][0]
            out = []
            for y in range(NP):
                src = tbl_hbm.at[pl.ds(pl.multiple_of(
                    (half * nhw + row_base + y * w) * CH, CH), NPCH)]
                dst = patch_v.at[pl.ds(y * NPCH, NPCH)]
                if make_only:
                    out.append(pltpu.make_async_copy(src, dst, sem))
                else:
                    out.append(pltpu.async_copy(src, dst, sem))
            return out

        def out_copy(pp):
            bx = jnp.right_shift(pp, 1)
            half = jnp.bitwise_and(pp, halves - 1)
            mm = wid * bpw + bx
            src = out_v
            dst = out_hbm.at[mm, pl.ds(half * (CH * NPIX), CH * NPIX)]
            return src, dst

        slab_copies(0, make_only=False)

        def pass_body(p, carry):
            bx = jnp.right_shift(p, 1)

            pltpu.make_async_copy(
                tbl_hbm.at[pl.ds(0, PWORDS)], patch_v, sem).wait()

            xi0 = reci_v[pl.ds(bx * 128, L)]
            xi1 = reci_v[pl.ds(bx * 128 + 16, L)]
            yi0 = reci_v[pl.ds(bx * 128 + 32, L)]
            yi1 = reci_v[pl.ds(bx * 128 + 48, L)]
            wxf0 = recf_v[pl.ds(bx * 128, L)]
            wxf1 = recf_v[pl.ds(bx * 128 + 16, L)]
            wyf0 = recf_v[pl.ds(bx * 128 + 32, L)]
            wyf1 = recf_v[pl.ds(bx * 128 + 48, L)]
            x0c = [xi0[j] for j in range(CROP)]
            x1c = [xi1[j] for j in range(CROP)]
            y0m = [yi0[j] for j in range(CROP)]
            y1m = [yi1[j] for j in range(CROP)]
            wx0 = [wxf0[j] for j in range(CROP)]
            wx1 = [wxf1[j] for j in range(CROP)]
            wy0 = [wyf0[j] for j in range(CROP)]
            wy1 = [wyf1[j] for j in range(CROP)]

            @plsc.parallel_loop(0, NP, unroll=1)
            def ay_body(y):
                pb = y * NPCH
                sb = y * CHXP
                for j in range(CROP):
                    for ch in range(CH // L):
                        v0 = patch_v[pl.ds(pb + x0c[j] + ch * L, L)]
                        v1 = patch_v[pl.ds(pb + x1c[j] + ch * L, L)]
                        plsc.store_scatter(
                            xrow_v,
                            [jnp.full((L,), sb + j + ch * (L * XP),
                                      jnp.int32) + lane_xp],
                            wx0[j] * v0 + wx1[j] * v1)

            @pl.when(p + 1 < total)
            def _():
                slab_copies(p + 1, make_only=False)

            @pl.when(p >= 1)
            def _():
                src, dst = out_copy(p - 1)
                pltpu.make_async_copy(src, dst, osem).wait()

            @plsc.parallel_loop(0, CH, unroll=8)
            def sb_body(cc):
                base = jnp.full((L,), cc * NPIX, jnp.int32) + lane
                off = cc * XP
                for i in range(CROP):
                    v0 = xrow_v[pl.ds(y0m[i] + off, L)]
                    v1 = xrow_v[pl.ds(y1m[i] + off, L)]
                    plsc.store_scatter(
                        out_v, [base + (i * CROP)],
                        wy0[i] * v0 + wy1[i] * v1, mask=jmask)

            src, dst = out_copy(p)
            pltpu.async_copy(src, dst, osem)
            return carry

        lax.fori_loop(0, total, pass_body, 0, unroll=False)

        src, dst = out_copy(jnp.int32(total - 1))
        pltpu.make_async_copy(src, dst, osem).wait()

    return k(tbl, rec_i, rec_f)


def kernel(featuremap, boxes, box_ind):
    n, c, h, w = featuremap.shape
    m = boxes.shape[0]
    nhw = n * h * w
    halves = c // CH

    tblh = jnp.transpose(featuremap, (0, 2, 3, 1)).reshape(nhw, halves, CH)
    tbl = jnp.transpose(tblh, (1, 0, 2)).reshape(halves * nhw * CH)

    x1, y1, x2, y2 = boxes[:, 0], boxes[:, 1], boxes[:, 2], boxes[:, 3]
    spacing_w = (x2 - x1) / CROP
    spacing_h = (y2 - y1) / CROP
    nx0 = (x1 + spacing_w / 2 - 0.5) / (w - 1)
    ny0 = (y1 + spacing_h / 2 - 0.5) / (h - 1)
    nw_ = spacing_w * (CROP - 1) / (w - 1)
    nh_ = spacing_h * (CROP - 1) / (h - 1)
    g = jnp.linspace(0.0, 1.0, CROP)
    iy = (ny0[:, None] + nh_[:, None] * g[None, :]) * (h - 1)
    ix = (nx0[:, None] + nw_[:, None] * g[None, :]) * (w - 1)
    iy0 = jnp.floor(iy)
    ix0 = jnp.floor(ix)
    wy1 = iy - iy0
    wx1 = ix - ix0
    vy0 = (iy0 >= 0) & (iy0 <= h - 1)
    vy1 = (iy0 + 1 >= 0) & (iy0 + 1 <= h - 1)
    vx0 = (ix0 >= 0) & (ix0 <= w - 1)
    vx1 = (ix0 + 1 >= 0) & (ix0 + 1 <= w - 1)
    wy0z = (1.0 - wy1) * vy0
    wy1z = wy1 * vy1
    wx0z = (1.0 - wx1) * vx0
    wx1z = wx1 * vx1
    ix0 = ix0.astype(jnp.int32)
    iy0 = iy0.astype(jnp.int32)
    xbase = jnp.clip(ix0[:, 0], 0, w - NXP)
    ybase = jnp.clip(iy0[:, 0], 0, h - NP)
    x0r = jnp.clip(ix0 - xbase[:, None], 0, NXP - 1) * CH
    x1r = jnp.clip(ix0 + 1 - xbase[:, None], 0, NXP - 1) * CH
    y0m = jnp.clip(iy0 - ybase[:, None], 0, NP - 1) * CHXP
    y1m = jnp.clip(iy0 + 1 - ybase[:, None], 0, NP - 1) * CHXP
    row_base = (box_ind.astype(jnp.int32) * h + ybase) * w + xbase

    def pad16(a):
        return jnp.pad(a, ((0, 0), (0, 16 - CROP)))

    rec_i = jnp.stack([
        pad16(x0r), pad16(x1r), pad16(y0m), pad16(y1m),
        jnp.broadcast_to(row_base[:, None], (m, 16)),
    ], axis=1).astype(jnp.int32).reshape(m, 80)
    rec_i = jnp.pad(rec_i, ((0, 0), (0, 48)))
    rec_f = jnp.stack(
        [pad16(wx0z), pad16(wx1z), pad16(wy0z), pad16(wy1z)], axis=1
    ).astype(jnp.float32).reshape(m, 64)
    rec_f = jnp.pad(rec_f, ((0, 0), (0, 64)))

    out = _sc_roialign(tbl, rec_i, rec_f, m=m, c=c, nhw=nhw, w=w)
    return out.reshape(m, c, CROP, CROP)


# speedup vs baseline: 1.0164x; 1.0164x over previous
import functools

import jax
import jax.numpy as jnp
from jax import lax
from jax.experimental import pallas as pl
from jax.experimental.pallas import tpu as pltpu
from jax.experimental.pallas import tpu_sc as plsc

CROP = 14
NPIX = CROP * CROP
NC, NS, L = 2, 16, 16
NW = NC * NS
NP = 17
NXP = 17
CH = 128
XP = 15
CHXP = CH * XP
NPCH = NXP * CH
PWORDS = NP * NPCH
OWORDS = CH * NPIX


def _sc_roialign(tbl, rec_i, rec_f, *, m, c, nhw, w):
    bpw = m // NW
    halves = c // CH
    total = bpw * halves

    @functools.partial(
        pl.kernel,
        out_type=jax.ShapeDtypeStruct((m, c * NPIX), jnp.float32),
        mesh=plsc.VectorSubcoreMesh(core_axis_name="c", subcore_axis_name="s"),
        scratch_types=[
            pltpu.VMEM((bpw * 128,), jnp.int32),
            pltpu.VMEM((bpw * 128,), jnp.float32),
            pltpu.VMEM((PWORDS,), jnp.float32),
            pltpu.VMEM((NP * CHXP + 16,), jnp.float32),
            pltpu.VMEM((CH * NPIX,), jnp.float32),
            pltpu.SemaphoreType.DMA,
            pltpu.SemaphoreType.DMA,
        ],
        compiler_params=pltpu.CompilerParams(needs_layout_passes=False),
    )
    def k(tbl_hbm, reci_hbm, recf_hbm, out_hbm,
          reci_v, recf_v, patch_v, xrow_v, out_v, sem, osem):
        wid = lax.axis_index("s") * NC + lax.axis_index("c")
        lane = lax.iota(jnp.int32, L)
        jmask = lane < CROP
        lane_xp = lane * XP

        for bb in range(bpw):
            pltpu.sync_copy(reci_hbm.at[wid * bpw + bb],
                            reci_v.at[pl.ds(bb * 128, 128)])
            pltpu.sync_copy(recf_hbm.at[wid * bpw + bb],
                            recf_v.at[pl.ds(bb * 128, 128)])

        def slab_copies(pp, make_only):
            bx = jnp.right_shift(pp, 1)
            half = jnp.bitwise_and(pp, halves - 1)
            row_base = reci_v[pl.ds(bx * 128 + 64, L)][0]
            out = []
            for y in range(NP):
                src = tbl_hbm.at[pl.ds(pl.multiple_of(
                    (half * nhw + row_base + y * w) * CH, CH), NPCH)]
                dst = patch_v.at[pl.ds(y * NPCH, NPCH)]
                if make_only:
                    out.append(pltpu.make_async_copy(src, dst, sem))
                else:
                    out.append(pltpu.async_copy(src, dst, sem))
            return out

        def out_copy(pp):
            bx = jnp.right_shift(pp, 1)
            half = jnp.bitwise_and(pp, halves - 1)
            mm = wid * bpw + bx
            src = out_v
            dst = out_hbm.at[mm, pl.ds(half * (CH * NPIX), CH * NPIX)]
            return src, dst

        slab_copies(0, make_only=False)

        def pass_body(p, carry):
            bx = jnp.right_shift(p, 1)

            pltpu.make_async_copy(
                tbl_hbm.at[pl.ds(0, PWORDS)], patch_v, sem).wait()

            xi0 = reci_v[pl.ds(bx * 128, L)]
            xi1 = reci_v[pl.ds(bx * 128 + 16, L)]
            yi0 = reci_v[pl.ds(bx * 128 + 32, L)]
            yi1 = reci_v[pl.ds(bx * 128 + 48, L)]
            wxf0 = recf_v[pl.ds(bx * 128, L)]
            wxf1 = recf_v[pl.ds(bx * 128 + 16, L)]
            wyf0 = recf_v[pl.ds(bx * 128 + 32, L)]
            wyf1 = recf_v[pl.ds(bx * 128 + 48, L)]
            x0c = [xi0[j] for j in range(CROP)]
            x1c = [xi1[j] for j in range(CROP)]
            y0m = [yi0[j] for j in range(CROP)]
            y1m = [yi1[j] for j in range(CROP)]
            wx0 = [wxf0[j] for j in range(CROP)]
            wx1 = [wxf1[j] for j in range(CROP)]
            wy0 = [wyf0[j] for j in range(CROP)]
            wy1 = [wyf1[j] for j in range(CROP)]

            @plsc.parallel_loop(0, NP, unroll=1)
            def ay_body(y):
                pb = y * NPCH
                sb = y * CHXP
                for j in range(CROP):
                    for ch in range(CH // L):
                        v0 = patch_v[pl.ds(pb + x0c[j] + ch * L, L)]
                        v1 = patch_v[pl.ds(pb + x1c[j] + ch * L, L)]
                        plsc.store_scatter(
                            xrow_v,
                            [jnp.full((L,), sb + j + ch * (L * XP),
                                      jnp.int32) + lane_xp],
                            wx0[j] * v0 + wx1[j] * v1)

            @pl.when(p + 1 < total)
            def _():
                slab_copies(p + 1, make_only=False)

            @pl.when(p >= 1)
            def _():
                src, dst = out_copy(p - 1)
                pltpu.make_async_copy(src, dst, osem).wait()

            @plsc.parallel_loop(0, CH, unroll=4)
            def sb_body(cc):
                base = jnp.full((L,), cc * NPIX, jnp.int32) + lane
                off = cc * XP
                for i in range(CROP):
                    v0 = xrow_v[pl.ds(y0m[i] + off, L)]
                    v1 = xrow_v[pl.ds(y1m[i] + off, L)]
                    plsc.store_scatter(
                        out_v, [base + (i * CROP)],
                        wy0[i] * v0 + wy1[i] * v1, mask=jmask)

            src, dst = out_copy(p)
            pltpu.async_copy(src, dst, osem)
            return carry

        lax.fori_loop(0, total, pass_body, 0, unroll=False)

        src, dst = out_copy(jnp.int32(total - 1))
        pltpu.make_async_copy(src, dst, osem).wait()

    return k(tbl, rec_i, rec_f)


def kernel(featuremap, boxes, box_ind):
    n, c, h, w = featuremap.shape
    m = boxes.shape[0]
    nhw = n * h * w
    halves = c // CH

    tblh = jnp.transpose(featuremap, (0, 2, 3, 1)).reshape(nhw, halves, CH)
    tbl = jnp.transpose(tblh, (1, 0, 2)).reshape(halves * nhw * CH)

    x1, y1, x2, y2 = boxes[:, 0], boxes[:, 1], boxes[:, 2], boxes[:, 3]
    spacing_w = (x2 - x1) / CROP
    spacing_h = (y2 - y1) / CROP
    nx0 = (x1 + spacing_w / 2 - 0.5) / (w - 1)
    ny0 = (y1 + spacing_h / 2 - 0.5) / (h - 1)
    nw_ = spacing_w * (CROP - 1) / (w - 1)
    nh_ = spacing_h * (CROP - 1) / (h - 1)
    g = jnp.linspace(0.0, 1.0, CROP)
    iy = (ny0[:, None] + nh_[:, None] * g[None, :]) * (h - 1)
    ix = (nx0[:, None] + nw_[:, None] * g[None, :]) * (w - 1)
    iy0 = jnp.floor(iy)
    ix0 = jnp.floor(ix)
    wy1 = iy - iy0
    wx1 = ix - ix0
    vy0 = (iy0 >= 0) & (iy0 <= h - 1)
    vy1 = (iy0 + 1 >= 0) & (iy0 + 1 <= h - 1)
    vx0 = (ix0 >= 0) & (ix0 <= w - 1)
    vx1 = (ix0 + 1 >= 0) & (ix0 + 1 <= w - 1)
    wy0z = (1.0 - wy1) * vy0
    wy1z = wy1 * vy1
    wx0z = (1.0 - wx1) * vx0
    wx1z = wx1 * vx1
    ix0 = ix0.astype(jnp.int32)
    iy0 = iy0.astype(jnp.int32)
    xbase = jnp.clip(ix0[:, 0], 0, w - NXP)
    ybase = jnp.clip(iy0[:, 0], 0, h - NP)
    x0r = jnp.clip(ix0 - xbase[:, None], 0, NXP - 1) * CH
    x1r = jnp.clip(ix0 + 1 - xbase[:, None], 0, NXP - 1) * CH
    y0m = jnp.clip(iy0 - ybase[:, None], 0, NP - 1) * CHXP
    y1m = jnp.clip(iy0 + 1 - ybase[:, None], 0, NP - 1) * CHXP
    row_base = (box_ind.astype(jnp.int32) * h + ybase) * w + xbase

    def pad16(a):
        return jnp.pad(a, ((0, 0), (0, 16 - CROP)))

    rec_i = jnp.stack([
        pad16(x0r), pad16(x1r), pad16(y0m), pad16(y1m),
        jnp.broadcast_to(row_base[:, None], (m, 16)),
    ], axis=1).astype(jnp.int32).reshape(m, 80)
    rec_i = jnp.pad(rec_i, ((0, 0), (0, 48)))
    rec_f = jnp.stack(
        [pad16(wx0z), pad16(wx1z), pad16(wy0z), pad16(wy1z)], axis=1
    ).astype(jnp.float32).reshape(m, 64)
    rec_f = jnp.pad(rec_f, ((0, 0), (0, 64)))

    out = _sc_roialign(tbl, rec_i, rec_f, m=m, c=c, nhw=nhw, w=w)
    return out.reshape(m, c, CROP, CROP)


# speedup vs baseline: 1.2832x; 1.2625x over previous
import functools

import jax
import jax.numpy as jnp
from jax import lax
from jax.experimental import pallas as pl
from jax.experimental.pallas import tpu as pltpu
from jax.experimental.pallas import tpu_sc as plsc

CROP = 14
NPIX = CROP * CROP
NC, NS, L = 2, 16, 16
NW = NC * NS
NP = 17
NXP = 17
CH = 128
XP = 15
CHXP = CH * XP
NPCH = NXP * CH
PWORDS = NP * NPCH
OWORDS = CH * NPIX


def _sc_roialign(tbl, rec_i, rec_f, *, m, c, nhw, w):
    bpw = m // NW
    halves = c // CH
    total = bpw * halves

    @functools.partial(
        pl.kernel,
        out_type=jax.ShapeDtypeStruct((m, c, NPIX), jnp.float32),
        mesh=plsc.VectorSubcoreMesh(core_axis_name="c", subcore_axis_name="s"),
        scratch_types=[
            pltpu.VMEM((bpw * 128,), jnp.int32),
            pltpu.VMEM((bpw * 128,), jnp.float32),
            pltpu.VMEM((PWORDS,), jnp.float32),
            pltpu.VMEM((NP * CHXP + 16,), jnp.float32),
            pltpu.VMEM((CH, NPIX), jnp.float32),
            pltpu.SemaphoreType.DMA,
            pltpu.SemaphoreType.DMA,
        ],
        compiler_params=pltpu.CompilerParams(needs_layout_passes=False),
    )
    def k(tbl_hbm, reci_hbm, recf_hbm, out_hbm,
          reci_v, recf_v, patch_v, xrow_v, out_v, sem, osem):
        wid = lax.axis_index("s") * NC + lax.axis_index("c")
        lane = lax.iota(jnp.int32, L)
        jmask = lane < CROP
        lane_xp = lane * XP

        for bb in range(bpw):
            pltpu.sync_copy(reci_hbm.at[wid * bpw + bb],
                            reci_v.at[pl.ds(bb * 128, 128)])
            pltpu.sync_copy(recf_hbm.at[wid * bpw + bb],
                            recf_v.at[pl.ds(bb * 128, 128)])

        def slab_copies(pp, make_only):
            bx = jnp.right_shift(pp, 1)
            half = jnp.bitwise_and(pp, halves - 1)
            row_base = reci_v[pl.ds(bx * 128 + 64, L)][0]
            out = []
            for y in range(NP):
                src = tbl_hbm.at[pl.ds(pl.multiple_of(
                    (half * nhw + row_base + y * w) * CH, CH), NPCH)]
                dst = patch_v.at[pl.ds(y * NPCH, NPCH)]
                if make_only:
                    out.append(pltpu.make_async_copy(src, dst, sem))
                else:
                    out.append(pltpu.async_copy(src, dst, sem))
            return out

        def out_copy(pp):
            bx = jnp.right_shift(pp, 1)
            half = jnp.bitwise_and(pp, halves - 1)
            mm = wid * bpw + bx
            src = out_v
            dst = out_hbm.at[mm, pl.ds(half * CH, CH)]
            return src, dst

        slab_copies(0, make_only=False)

        def pass_body(p, carry):
            bx = jnp.right_shift(p, 1)

            pltpu.make_async_copy(
                tbl_hbm.at[pl.ds(0, PWORDS)], patch_v, sem).wait()

            xi0 = reci_v[pl.ds(bx * 128, L)]
            xi1 = reci_v[pl.ds(bx * 128 + 16, L)]
            yi0 = reci_v[pl.ds(bx * 128 + 32, L)]
            yi1 = reci_v[pl.ds(bx * 128 + 48, L)]
            wxf0 = recf_v[pl.ds(bx * 128, L)]
            wxf1 = recf_v[pl.ds(bx * 128 + 16, L)]
            wyf0 = recf_v[pl.ds(bx * 128 + 32, L)]
            wyf1 = recf_v[pl.ds(bx * 128 + 48, L)]
            x0c = [xi0[j] for j in range(CROP)]
            x1c = [xi1[j] for j in range(CROP)]
            y0m = [yi0[j] for j in range(CROP)]
            y1m = [yi1[j] for j in range(CROP)]
            wx0 = [wxf0[j] for j in range(CROP)]
            wx1 = [wxf1[j] for j in range(CROP)]
            wy0 = [wyf0[j] for j in range(CROP)]
            wy1 = [wyf1[j] for j in range(CROP)]

            @plsc.parallel_loop(0, NP, unroll=1)
            def ay_body(y):
                pb = y * NPCH
                sb = y * CHXP
                for j in range(CROP):
                    for ch in range(CH // L):
                        v0 = patch_v[pl.ds(pb + x0c[j] + ch * L, L)]
                        v1 = patch_v[pl.ds(pb + x1c[j] + ch * L, L)]
                        plsc.store_scatter(
                            xrow_v,
                            [jnp.full((L,), sb + j + ch * (L * XP),
                                      jnp.int32) + lane_xp],
                            wx0[j] * v0 + wx1[j] * v1)

            @pl.when(p + 1 < total)
            def _():
                slab_copies(p + 1, make_only=False)

            @pl.when(p >= 1)
            def _():
                src, dst = out_copy(p - 1)
                pltpu.make_async_copy(src, dst, osem).wait()

            @plsc.parallel_loop(0, CH, unroll=4)
            def sb_body(cc):
                ccv = jnp.full((L,), cc, jnp.int32)
                off = cc * XP
                for i in range(CROP):
                    v0 = xrow_v[pl.ds(y0m[i] + off, L)]
                    v1 = xrow_v[pl.ds(y1m[i] + off, L)]
                    plsc.store_scatter(
                        out_v,
                        [ccv,
                         jnp.full((L,), i * CROP, jnp.int32) + lane],
                        wy0[i] * v0 + wy1[i] * v1, mask=jmask)

            src, dst = out_copy(p)
            pltpu.async_copy(src, dst, osem)
            return carry

        lax.fori_loop(0, total, pass_body, 0, unroll=False)

        src, dst = out_copy(jnp.int32(total - 1))
        pltpu.make_async_copy(src, dst, osem).wait()

    return k(tbl, rec_i, rec_f)


def kernel(featuremap, boxes, box_ind):
    n, c, h, w = featuremap.shape
    m = boxes.shape[0]
    nhw = n * h * w
    halves = c // CH

    tblh = jnp.transpose(featuremap, (0, 2, 3, 1)).reshape(nhw, halves, CH)
    tbl = jnp.transpose(tblh, (1, 0, 2)).reshape(halves * nhw * CH)

    x1, y1, x2, y2 = boxes[:, 0], boxes[:, 1], boxes[:, 2], boxes[:, 3]
    spacing_w = (x2 - x1) / CROP
    spacing_h = (y2 - y1) / CROP
    nx0 = (x1 + spacing_w / 2 - 0.5) / (w - 1)
    ny0 = (y1 + spacing_h / 2 - 0.5) / (h - 1)
    nw_ = spacing_w * (CROP - 1) / (w - 1)
    nh_ = spacing_h * (CROP - 1) / (h - 1)
    g = jnp.linspace(0.0, 1.0, CROP)
    iy = (ny0[:, None] + nh_[:, None] * g[None, :]) * (h - 1)
    ix = (nx0[:, None] + nw_[:, None] * g[None, :]) * (w - 1)
    iy0 = jnp.floor(iy)
    ix0 = jnp.floor(ix)
    wy1 = iy - iy0
    wx1 = ix - ix0
    vy0 = (iy0 >= 0) & (iy0 <= h - 1)
    vy1 = (iy0 + 1 >= 0) & (iy0 + 1 <= h - 1)
    vx0 = (ix0 >= 0) & (ix0 <= w - 1)
    vx1 = (ix0 + 1 >= 0) & (ix0 + 1 <= w - 1)
    wy0z = (1.0 - wy1) * vy0
    wy1z = wy1 * vy1
    wx0z = (1.0 - wx1) * vx0
    wx1z = wx1 * vx1
    ix0 = ix0.astype(jnp.int32)
    iy0 = iy0.astype(jnp.int32)
    xbase = jnp.clip(ix0[:, 0], 0, w - NXP)
    ybase = jnp.clip(iy0[:, 0], 0, h - NP)
    x0r = jnp.clip(ix0 - xbase[:, None], 0, NXP - 1) * CH
    x1r = jnp.clip(ix0 + 1 - xbase[:, None], 0, NXP - 1) * CH
    y0m = jnp.clip(iy0 - ybase[:, None], 0, NP - 1) * CHXP
    y1m = jnp.clip(iy0 + 1 - ybase[:, None], 0, NP - 1) * CHXP
    row_base = (box_ind.astype(jnp.int32) * h + ybase) * w + xbase

    def pad16(a):
        return jnp.pad(a, ((0, 0), (0, 16 - CROP)))

    rec_i = jnp.stack([
        pad16(x0r), pad16(x1r), pad16(y0m), pad16(y1m),
        jnp.broadcast_to(row_base[:, None], (m, 16)),
    ], axis=1).astype(jnp.int32).reshape(m, 80)
    rec_i = jnp.pad(rec_i, ((0, 0), (0, 48)))
    rec_f = jnp.stack(
        [pad16(wx0z), pad16(wx1z), pad16(wy0z), pad16(wy1z)], axis=1
    ).astype(jnp.float32).reshape(m, 64)
    rec_f = jnp.pad(rec_f, ((0, 0), (0, 64)))

    out = _sc_roialign(tbl, rec_i, rec_f, m=m, c=c, nhw=nhw, w=w)
    return out.reshape(m, c, CROP, CROP)
